# baseline (speedup 1.0000x reference)
import functools

import jax
import jax.numpy as jnp
from jax import lax
from jax.experimental import pallas as pl
from jax.experimental.pallas import tpu as pltpu
from jax.experimental.pallas import tpu_sc as plsc

NC = 2
NS = 16
NW = NC * NS

IDX_W = 100
CHUNK_SEQ = 2


def _sc_embed(tokens2d, table, pe, B, L, E):
    n_rows = B * L
    rows_per_w = n_rows // NW
    seq_per_w = B // NW
    chunk_rows = CHUNK_SEQ * L
    n_chunks = seq_per_w // CHUNK_SEQ
    idx_rows = chunk_rows // IDX_W
    tok_rows_per_w = rows_per_w // IDX_W

    mesh = plsc.VectorSubcoreMesh(core_axis_name="c", subcore_axis_name="s")

    @functools.partial(
        pl.kernel,
        mesh=mesh,
        out_type=jax.ShapeDtypeStruct((n_rows, E), jnp.float32),
        scratch_types=[
            pltpu.VMEM((idx_rows, IDX_W), jnp.int32),
            pltpu.VMEM((chunk_rows, E), jnp.float32),
            pltpu.VMEM((L, E), jnp.float32),
            pltpu.SemaphoreType.DMA,
        ],
        compiler_params=pltpu.CompilerParams(use_tc_tiling_on_sc=False),
    )
    def k(tok_hbm, table_hbm, pe_hbm, out_hbm, idx_v, rows_v, pe_v, sem):
        wid = lax.axis_index("s") * NC + lax.axis_index("c")
        pltpu.sync_copy(pe_hbm, pe_v)
        base_row = wid * rows_per_w
        tok_base = wid * tok_rows_per_w

        def chunk_body(c, carry):
            row0 = base_row + c * chunk_rows
            pltpu.sync_copy(
                tok_hbm.at[pl.ds(tok_base + c * idx_rows, idx_rows)], idx_v
            )
            copies = [
                pltpu.async_copy(
                    table_hbm.at[idx_v.at[j]],
                    rows_v.at[pl.ds(j * IDX_W, IDX_W)],
                    sem,
                )
                for j in range(idx_rows)
            ]
            for cp in copies:
                cp.wait()
            for s in range(CHUNK_SEQ):
                def add_body(i, acc):
                    for q in range(E // 16):
                        sl = pl.ds(q * 16, 16)
                        rows_v[s * L + i, sl] = rows_v[s * L + i, sl] + pe_v[i, sl]
                    return acc
                lax.fori_loop(0, L, add_body, 0)
            pltpu.sync_copy(rows_v, out_hbm.at[pl.ds(row0, chunk_rows)])
            return carry

        lax.fori_loop(0, n_chunks, chunk_body, 0)

    return k(tokens2d, table, pe)


def kernel(tokens, table, pe):
    B, L = tokens.shape
    E = table.shape[1]
    tok2d = tokens.reshape(B * L // IDX_W, IDX_W)
    out = _sc_embed(tok2d, table, pe[:L], B, L, E)
    return out.reshape(B, L, E)


# speedup vs baseline: 1.1151x; 1.1151x over previous
import functools

import jax
import jax.numpy as jnp
from jax import lax
from jax.experimental import pallas as pl
from jax.experimental.pallas import tpu as pltpu
from jax.experimental.pallas import tpu_sc as plsc

NC = 2
NS = 16
NW = NC * NS

IDX_W = 100
CHUNK_SEQ = 2


def _sc_embed(tokens2d, table, pe, ident, B, L, E):
    n_rows = B * L
    rows_per_w = n_rows // NW
    seq_per_w = B // NW
    chunk_rows = CHUNK_SEQ * L
    n_chunks = seq_per_w // CHUNK_SEQ
    idx_rows = chunk_rows // IDX_W
    pe_rows = L // IDX_W
    tok_rows_per_w = rows_per_w // IDX_W
    n_pairs = n_chunks // 2

    mesh = plsc.VectorSubcoreMesh(core_axis_name="c", subcore_axis_name="s")

    @functools.partial(
        pl.kernel,
        mesh=mesh,
        out_type=jax.ShapeDtypeStruct((n_rows, E), jnp.float32),
        scratch_types=[
            pltpu.VMEM((idx_rows, IDX_W), jnp.int32),
            pltpu.VMEM((idx_rows, IDX_W), jnp.int32),
            pltpu.VMEM((chunk_rows, E), jnp.float32),
            pltpu.VMEM((chunk_rows, E), jnp.float32),
            pltpu.VMEM((L, E), jnp.float32),
            pltpu.VMEM((idx_rows, IDX_W), jnp.int32),
            pltpu.SemaphoreType.DMA,
            pltpu.SemaphoreType.DMA,
            pltpu.SemaphoreType.DMA,
            pltpu.SemaphoreType.DMA,
            pltpu.SemaphoreType.DMA,
            pltpu.SemaphoreType.DMA,
        ],
        compiler_params=pltpu.CompilerParams(use_tc_tiling_on_sc=False),
    )
    def k(tok_hbm, table_hbm, pe_hbm, ident_hbm, out_hbm,
          idx_a, idx_b, rows_a, rows_b, pe_v, ident_v,
          sem_ia, sem_ib, sem_ga, sem_gb, sem_oa, sem_ob):
        wid = lax.axis_index("s") * NC + lax.axis_index("c")
        pltpu.sync_copy(pe_hbm, pe_v)
        pltpu.sync_copy(ident_hbm, ident_v)
        base_row = wid * rows_per_w
        tok_base = wid * tok_rows_per_w

        def tok_src(c):
            return tok_hbm.at[pl.ds(tok_base + c * idx_rows, idx_rows)]

        def out_dst(c):
            return out_hbm.at[pl.ds(base_row + c * chunk_rows, chunk_rows)]

        def fire_idx(c, ib, sem):
            pltpu.async_copy(tok_src(c), ib, sem)

        def wait_idx(c, ib, sem):
            pltpu.make_async_copy(tok_src(c), ib, sem).wait()

        def fire_gather(ib, rb, sem):
            for j in range(idx_rows):
                pltpu.async_copy(
                    table_hbm.at[ib.at[j]], rb.at[pl.ds(j * IDX_W, IDX_W)], sem
                )

        def wait_gather(ib, rb, sem):
            for j in range(idx_rows):
                pltpu.make_async_copy(
                    table_hbm.at[ib.at[j]], rb.at[pl.ds(j * IDX_W, IDX_W)], sem
                ).wait()

        def add_pe(rb):
            for s in range(CHUNK_SEQ):
                def body(r, carry):
                    for e in range(E // 16):
                        sl = pl.ds(e * 16, 16)
                        rb[s * L + r, sl] = rb[s * L + r, sl] + pe_v[r, sl]
                    return carry

                lax.fori_loop(0, L, body, 0)

        def fire_store(c, rb, sem):
            pltpu.async_copy(rb, out_dst(c), sem)

        def wait_store(c, rb, sem):
            pltpu.make_async_copy(rb, out_dst(c), sem).wait()

        fire_idx(0, idx_a, sem_ia)
        fire_idx(1, idx_b, sem_ib)

        def pair_body(t, carry):
            ca = 2 * t
            cb = ca + 1
            wait_idx(ca, idx_a, sem_ia)

            @pl.when(t > 0)
            def _():
                wait_store(ca - 2, rows_a, sem_oa)

            fire_gather(idx_a, rows_a, sem_ga)

            @pl.when(t > 0)
            def _():
                wait_gather(idx_b, rows_b, sem_gb)
                fire_idx(cb, idx_b, sem_ib)
                add_pe(rows_b)
                fire_store(cb - 2, rows_b, sem_ob)

            wait_idx(cb, idx_b, sem_ib)

            @pl.when(t > 0)
            def _():
                wait_store(cb - 2, rows_b, sem_ob)

            fire_gather(idx_b, rows_b, sem_gb)

            wait_gather(idx_a, rows_a, sem_ga)

            @pl.when(ca + 2 < n_chunks)
            def _():
                fire_idx(ca + 2, idx_a, sem_ia)

            add_pe(rows_a)
            fire_store(ca, rows_a, sem_oa)
            return carry

        lax.fori_loop(0, n_pairs, pair_body, 0)

        c_last = n_chunks - 1
        wait_gather(idx_b, rows_b, sem_gb)
        add_pe(rows_b)
        fire_store(c_last, rows_b, sem_ob)
        wait_store(n_chunks - 2, rows_a, sem_oa)
        wait_store(c_last, rows_b, sem_ob)

    return k(tokens2d, table, pe, ident)


def kernel(tokens, table, pe):
    B, L = tokens.shape
    E = table.shape[1]
    chunk_rows = CHUNK_SEQ * L
    tok2d = tokens.reshape(B * L // IDX_W, IDX_W)
    ident = jnp.arange(chunk_rows, dtype=jnp.int32).reshape(
        chunk_rows // IDX_W, IDX_W
    )
    out = _sc_embed(tok2d, table, pe[:L], ident, B, L, E)
    return out.reshape(B, L, E)


# speedup vs baseline: 1.1519x; 1.0330x over previous
import functools

import jax
import jax.numpy as jnp
from jax import lax
from jax.experimental import pallas as pl
from jax.experimental.pallas import tpu as pltpu
from jax.experimental.pallas import tpu_sc as plsc

NC = 2
NS = 16
NW = NC * NS

IDX_W = 100
CHUNK_SEQ = 4


def _sc_embed(tokens2d, table, pe, B, L, E):
    n_rows = B * L
    rows_per_w = n_rows // NW
    seq_per_w = B // NW
    chunk_rows = CHUNK_SEQ * L
    n_chunks = seq_per_w // CHUNK_SEQ
    idx_rows = chunk_rows // IDX_W
    tok_rows_per_w = rows_per_w // IDX_W
    n_pairs = n_chunks // 2

    mesh = plsc.VectorSubcoreMesh(core_axis_name="c", subcore_axis_name="s")

    @functools.partial(
        pl.kernel,
        mesh=mesh,
        out_type=jax.ShapeDtypeStruct((n_rows, E), jnp.float32),
        scratch_types=[
            pltpu.VMEM((idx_rows, IDX_W), jnp.int32),
            pltpu.VMEM((idx_rows, IDX_W), jnp.int32),
            pltpu.VMEM((chunk_rows, E), jnp.float32),
            pltpu.VMEM((chunk_rows, E), jnp.float32),
            pltpu.VMEM((L, E), jnp.float32),
            pltpu.SemaphoreType.DMA,
            pltpu.SemaphoreType.DMA,
            pltpu.SemaphoreType.DMA,
            pltpu.SemaphoreType.DMA,
            pltpu.SemaphoreType.DMA,
            pltpu.SemaphoreType.DMA,
        ],
        compiler_params=pltpu.CompilerParams(use_tc_tiling_on_sc=False),
    )
    def k(tok_hbm, table_hbm, pe_hbm, out_hbm,
          idx_a, idx_b, rows_a, rows_b, pe_v,
          sem_ia, sem_ib, sem_ga, sem_gb, sem_oa, sem_ob):
        wid = lax.axis_index("s") * NC + lax.axis_index("c")
        pltpu.sync_copy(pe_hbm, pe_v)
        base_row = wid * rows_per_w
        tok_base = wid * tok_rows_per_w

        def tok_src(c):
            return tok_hbm.at[pl.ds(tok_base + c * idx_rows, idx_rows)]

        def out_dst(c):
            return out_hbm.at[pl.ds(base_row + c * chunk_rows, chunk_rows)]

        def fire_idx(c, ib, sem):
            pltpu.async_copy(tok_src(c), ib, sem)

        def wait_idx(c, ib, sem):
            pltpu.make_async_copy(tok_src(c), ib, sem).wait()

        def fire_gather(ib, rb, sem):
            for j in range(idx_rows):
                pltpu.async_copy(
                    table_hbm.at[ib.at[j]], rb.at[pl.ds(j * IDX_W, IDX_W)], sem
                )

        def wait_gather(ib, rb, sem):
            for j in range(idx_rows):
                pltpu.make_async_copy(
                    table_hbm.at[ib.at[j]], rb.at[pl.ds(j * IDX_W, IDX_W)], sem
                ).wait()

        def add_pe(rb):
            def body(r, carry):
                for e in range(E // 16):
                    sl = pl.ds(e * 16, 16)
                    p = pe_v[r, sl]
                    for s in range(CHUNK_SEQ):
                        rb[s * L + r, sl] = rb[s * L + r, sl] + p
                return carry

            lax.fori_loop(0, L, body, 0, unroll=2)

        def fire_store(c, rb, sem):
            pltpu.async_copy(rb, out_dst(c), sem)

        def wait_store(c, rb, sem):
            pltpu.make_async_copy(rb, out_dst(c), sem).wait()

        fire_idx(0, idx_a, sem_ia)
        fire_idx(1, idx_b, sem_ib)

        def pair_body(t, carry):
            ca = 2 * t
            cb = ca + 1
            wait_idx(ca, idx_a, sem_ia)

            @pl.when(t > 0)
            def _():
                wait_store(ca - 2, rows_a, sem_oa)

            fire_gather(idx_a, rows_a, sem_ga)

            @pl.when(t > 0)
            def _():
                wait_gather(idx_b, rows_b, sem_gb)
                fire_idx(cb, idx_b, sem_ib)
                add_pe(rows_b)
                fire_store(cb - 2, rows_b, sem_ob)

            wait_idx(cb, idx_b, sem_ib)

            @pl.when(t > 0)
            def _():
                wait_store(cb - 2, rows_b, sem_ob)

            fire_gather(idx_b, rows_b, sem_gb)

            wait_gather(idx_a, rows_a, sem_ga)

            @pl.when(ca + 2 < n_chunks)
            def _():
                fire_idx(ca + 2, idx_a, sem_ia)

            add_pe(rows_a)
            fire_store(ca, rows_a, sem_oa)
            return carry

        lax.fori_loop(0, n_pairs, pair_body, 0)

        c_last = n_chunks - 1
        wait_gather(idx_b, rows_b, sem_gb)
        add_pe(rows_b)
        fire_store(c_last, rows_b, sem_ob)
        wait_store(n_chunks - 2, rows_a, sem_oa)
        wait_store(c_last, rows_b, sem_ob)

    return k(tokens2d, table, pe)


def kernel(tokens, table, pe):
    B, L = tokens.shape
    E = table.shape[1]
    tok2d = tokens.reshape(B * L // IDX_W, IDX_W)
    out = _sc_embed(tok2d, table, pe[:L], B, L, E)
    return out.reshape(B, L, E)
